# Initial kernel scaffold; baseline (speedup 1.0000x reference)
import jax
import jax.numpy as jnp
from jax.experimental import pallas as pl


def kernel(src_pts, tgt_pts, R_init, t_init, W_fe1, b_fe1, W_fe2, b_fe2, w_wl, W_ds1, b_ds1, W_ds2, b_ds2, W_dt1, b_dt1, W_dt2, b_dt2):
    raise NotImplementedError("write your pallas kernel here")


# speedup vs baseline: 1.9099x; 1.9099x over previous
import functools

import jax
import jax.numpy as jnp
from jax import lax
from jax.experimental import pallas as pl
from jax.experimental.pallas import tpu as pltpu
from jax.experimental.pallas import tpu_sc as plsc

B, N, F = 2, 2048, 32
K, NS, C = 64, 32, 216
Q = K * C
QCH = 768
NCH = Q // QCH
DT = 128
ROWS = B * Q
NWORK = 32
RPW = ROWS // NWORK
GCH = 72
NGC = RPW // GCH

_HI = lax.Precision.HIGHEST
_DEF = lax.Precision.DEFAULT


def _mm(a, b, prec):
    return lax.dot_general(a, b, (((1,), (0,)), ((), ())), precision=prec)


def _prep_body(sptsT_ref, tptsT_ref, spts_ref, tpts_ref, RT_ref,
               Wf1_ref, bf1_ref, Wf2_ref, bf2_ref, wwl_ref,
               Wd1_ref, bd1_ref, Wd2_ref, bd2_ref, off_ref,
               keypts_ref, trans_ref, nn_ref, dfe_ref, table_ref):
    b = pl.program_id(0)
    sT = sptsT_ref[0]
    tT = tptsT_ref[0]
    sp = spts_ref[0]
    tp = tpts_ref[0]
    W1 = Wf1_ref[...]
    b1 = bf1_ref[...]
    W2 = Wf2_ref[...]
    b2 = bf2_ref[...]

    src_feat = jax.nn.relu(_mm(jax.nn.relu(_mm(sT, W1, _DEF) + b1), W2, _DEF) + b2)
    tgt_feat = jax.nn.relu(_mm(jax.nn.relu(_mm(tT, W1, _DEF) + b1), W2, _DEF) + b2)
    scores = _mm(src_feat, wwl_ref[...], _DEF)

    col_iota_n = lax.broadcasted_iota(jnp.int32, (N, 1), 0)
    lane_kn = lax.broadcasted_iota(jnp.int32, (K, N), 1)
    row_kn = lax.broadcasted_iota(jnp.int32, (K, N), 0)

    def topk_body(j, carry):
        sc, sel = carry
        m = jnp.max(sc)
        idx = jnp.min(jnp.where(sc == m, col_iota_n, N))
        sel = sel + jnp.where((row_kn == j) & (lane_kn == idx), 1.0, 0.0)
        sc = jnp.where(col_iota_n == idx, -jnp.inf, sc)
        return sc, sel

    _, sel = lax.fori_loop(0, K, topk_body,
                           (scores, jnp.zeros((K, N), jnp.float32)))

    keypts = _mm(sel, sT, _HI)
    key_feat = _mm(sel, src_feat, _HI)
    keyptsT = lax.dot_general(sp, sel, (((1,), (1,)), ((), ())),
                              precision=_HI)

    dx = keypts[:, 0:1] - keyptsT[0:1, :]
    dy = keypts[:, 1:2] - keyptsT[1:2, :]
    dz = keypts[:, 2:3] - keyptsT[2:3, :]
    d2 = (dx * dx + dy * dy) + dz * dz

    col_kk = lax.broadcasted_iota(jnp.int32, (K, K), 1)
    rowmod = lax.broadcasted_iota(jnp.int32, (K * NS, 1), 0) % NS

    def knn_body(s, carry):
        d2w, G = carry
        rm = jnp.min(d2w, axis=1, keepdims=True)
        idxc = jnp.min(jnp.where(d2w == rm, col_kk, K), axis=1, keepdims=True)
        oh = jnp.where(col_kk == idxc, 1.0, 0.0)
        exp_oh = jnp.broadcast_to(oh[:, None, :], (K, NS, K)).reshape(K * NS, K)
        G = G + jnp.where(rowmod == s, exp_oh, 0.0)
        d2w = jnp.where(col_kk == idxc, jnp.inf, d2w)
        return d2w, G

    _, G = lax.fori_loop(0, NS, knn_body,
                         (d2, jnp.zeros((K * NS, K), jnp.float32)))

    gxyz = _mm(G, keypts, _HI)
    gfeat = _mm(G, key_feat, _HI)
    kp_rep = jnp.broadcast_to(keypts[:, None, :], (K, NS, 3)).reshape(K * NS, 3)
    cat = jnp.concatenate([gxyz - kp_rep, gfeat], axis=1)
    hs = jax.nn.relu(_mm(cat, Wd1_ref[...], _DEF) + bd1_ref[...])
    hs = jax.nn.relu(_mm(hs, Wd2_ref[...], _DEF) + bd2_ref[...])
    dfe_ref[0] = jnp.max(hs.reshape(K, NS, F), axis=1)

    trans = _mm(keypts, RT_ref[...], _DEF)
    cand = trans[:, None, :] + off_ref[...][None, :, :]
    cf = cand.reshape(Q, 3)
    keypts_ref[0] = keypts
    trans_ref[0] = trans

    table_ref[0] = jnp.concatenate(
        [tT, tgt_feat, jnp.zeros((N, DT - 3 - F), jnp.float32)], axis=1)

    tx = tp[0:1, :]
    ty = tp[1:2, :]
    tz = tp[2:3, :]
    tn = (tx * tx + ty * ty) + tz * tz
    lane_qn = lax.broadcasted_iota(jnp.int32, (QCH, N), 1)
    boff = b * N
    for t in range(NCH):
        cfc = cf[t * QCH:(t + 1) * QCH, :]
        qn = (cfc[:, 0:1] * cfc[:, 0:1] + cfc[:, 1:2] * cfc[:, 1:2]) \
            + cfc[:, 2:3] * cfc[:, 2:3]
        cross = _mm(cfc, tp, _DEF)
        d2t = (qn + tn) - 2.0 * cross
        m = jnp.min(d2t, axis=1, keepdims=True)
        idx = jnp.min(jnp.where(d2t == m, lane_qn, N), axis=1, keepdims=True)
        nn_ref[0, t * QCH:(t + 1) * QCH, :] = idx + boff


KB = 16
QB = KB * C


def _cpg_body(g_ref, trans_ref, dfe_ref, off_ref, Wt1_ref, bt1_ref, Wt2_ref,
              bt2_ref, vcp_ref):
    g = g_ref[0]
    trans = trans_ref[0]
    cand = trans[:, None, :] + off_ref[...][None, :, :]
    cf = cand.reshape(QB, 3)
    tcat = jnp.concatenate([cf - g[:, 0:3], g[:, 3:3 + F]], axis=1)
    ht = jax.nn.relu(_mm(tcat, Wt1_ref[...], _DEF) + bt1_ref[...])
    ht = jax.nn.relu(_mm(ht, Wt2_ref[...], _DEF) + bt2_ref[...])
    dfe = dfe_ref[0]
    sim = jnp.sum(ht.reshape(KB, C, F) * dfe[:, None, :], axis=2)
    m = jnp.max(sim, axis=1, keepdims=True)
    e = jnp.exp(sim - m)
    w = e / jnp.sum(e, axis=1, keepdims=True)
    vcp_ref[0] = jnp.sum(w[:, :, None] * cand, axis=1)


_PREP_IN_SPECS = [
    pl.BlockSpec((1, N, 3), lambda b: (b, 0, 0)),
    pl.BlockSpec((1, N, 3), lambda b: (b, 0, 0)),
    pl.BlockSpec((1, 3, N), lambda b: (b, 0, 0)),
    pl.BlockSpec((1, 3, N), lambda b: (b, 0, 0)),
    pl.BlockSpec((3, 3), lambda b: (0, 0)),
    pl.BlockSpec((3, F), lambda b: (0, 0)),
    pl.BlockSpec((1, F), lambda b: (0, 0)),
    pl.BlockSpec((F, F), lambda b: (0, 0)),
    pl.BlockSpec((1, F), lambda b: (0, 0)),
    pl.BlockSpec((F, 1), lambda b: (0, 0)),
    pl.BlockSpec((3 + F, 64), lambda b: (0, 0)),
    pl.BlockSpec((1, 64), lambda b: (0, 0)),
    pl.BlockSpec((64, F), lambda b: (0, 0)),
    pl.BlockSpec((1, F), lambda b: (0, 0)),
    pl.BlockSpec((C, 3), lambda b: (0, 0)),
]
_PREP_OUT_SPECS = [
    pl.BlockSpec((1, K, 3), lambda b: (b, 0, 0)),
    pl.BlockSpec((1, K, 3), lambda b: (b, 0, 0)),
    pl.BlockSpec((1, Q, 1), lambda b: (b, 0, 0)),
    pl.BlockSpec((1, K, F), lambda b: (b, 0, 0)),
    pl.BlockSpec((1, N, DT), lambda b: (b, 0, 0)),
]
_PREP_OUT_SHAPES = [
    jax.ShapeDtypeStruct((B, K, 3), jnp.float32),
    jax.ShapeDtypeStruct((B, K, 3), jnp.float32),
    jax.ShapeDtypeStruct((B, Q, 1), jnp.int32),
    jax.ShapeDtypeStruct((B, K, F), jnp.float32),
    jax.ShapeDtypeStruct((B, N, DT), jnp.float32),
]

_CPG_IN_SPECS = [
    pl.BlockSpec((1, QB, DT), lambda b, kc: (b, kc, 0)),
    pl.BlockSpec((1, KB, 3), lambda b, kc: (b, kc, 0)),
    pl.BlockSpec((1, KB, F), lambda b, kc: (b, kc, 0)),
    pl.BlockSpec((C, 3), lambda b, kc: (0, 0)),
    pl.BlockSpec((3 + F, 64), lambda b, kc: (0, 0)),
    pl.BlockSpec((1, 64), lambda b, kc: (0, 0)),
    pl.BlockSpec((64, F), lambda b, kc: (0, 0)),
    pl.BlockSpec((1, F), lambda b, kc: (0, 0)),
]
_CPG_OUT_SPEC = pl.BlockSpec((1, KB, 3), lambda b, kc: (b, kc, 0))
_CPG_OUT_SHAPE = jax.ShapeDtypeStruct((B, K, 3), jnp.float32)


@functools.cache
def _make_sc_gather():
    @functools.partial(
        pl.kernel,
        out_type=jax.ShapeDtypeStruct((ROWS, DT), jnp.float32),
        mesh=plsc.VectorSubcoreMesh(core_axis_name="c", subcore_axis_name="s"),
        scratch_types=[
            pltpu.VMEM((RPW,), jnp.int32),
            pltpu.VMEM((RPW, DT), jnp.float32),
            pltpu.SemaphoreType.DMA,
        ],
    )
    def _sc_gather(table_hbm, idx_hbm, out_hbm, idx_v, rows_v, sem):
        wid = lax.axis_index("s") * 2 + lax.axis_index("c")
        base = wid * RPW
        pltpu.sync_copy(idx_hbm.at[pl.ds(base, RPW)], idx_v)
        copies = [
            pltpu.async_copy(table_hbm.at[idx_v.at[pl.ds(j * GCH, GCH)]],
                             rows_v.at[pl.ds(j * GCH, GCH)], sem)
            for j in range(NGC)
        ]
        for cp in copies:
            cp.wait()
        pltpu.sync_copy(rows_v, out_hbm.at[pl.ds(base, RPW)])

    return _sc_gather


def kernel(src_pts, tgt_pts, R_init, t_init, W_fe1, b_fe1, W_fe2, b_fe2,
           w_wl, W_ds1, b_ds1, W_ds2, b_ds2, W_dt1, b_dt1, W_dt2, b_dt2):
    sptsT = jnp.transpose(src_pts, (0, 2, 1))
    tptsT = jnp.transpose(tgt_pts, (0, 2, 1))
    g = jnp.linspace(-1.0, 1.0, 6)
    off = jnp.stack(jnp.meshgrid(g, g, g, indexing="ij"), -1).reshape(-1, 3)

    keypts, trans, nn, dfe, table = pl.pallas_call(
        _prep_body,
        grid=(B,),
        in_specs=_PREP_IN_SPECS,
        out_specs=_PREP_OUT_SPECS,
        out_shape=_PREP_OUT_SHAPES,
    )(sptsT, tptsT, src_pts, tgt_pts, R_init.T,
      W_fe1, b_fe1.reshape(1, F), W_fe2, b_fe2.reshape(1, F), w_wl,
      W_ds1, b_ds1.reshape(1, 64), W_ds2, b_ds2.reshape(1, F), off)

    gathered = _make_sc_gather()(table.reshape(B * N, DT), nn.reshape(ROWS))

    vcp = pl.pallas_call(
        _cpg_body,
        grid=(B, K // KB),
        in_specs=_CPG_IN_SPECS,
        out_specs=_CPG_OUT_SPEC,
        out_shape=_CPG_OUT_SHAPE,
    )(gathered.reshape(B, Q, DT), trans, dfe, off,
      W_dt1, b_dt1.reshape(1, 64), W_dt2, b_dt2.reshape(1, F))

    return keypts, vcp


# speedup vs baseline: 1.9124x; 1.0013x over previous
import functools

import jax
import jax.numpy as jnp
from jax import lax
from jax.experimental import pallas as pl
from jax.experimental.pallas import tpu as pltpu
from jax.experimental.pallas import tpu_sc as plsc

B, N, F = 2, 2048, 32
K, NS, C = 64, 32, 216
Q = K * C
QCH = 768
NCH = Q // QCH
DT = 128
ROWS = B * Q
NWORK = 32
RPW = ROWS // NWORK
GCH = 96
NGC = RPW // GCH

_HI = lax.Precision.HIGHEST
_DEF = lax.Precision.DEFAULT


def _mm(a, b, prec):
    return lax.dot_general(a, b, (((1,), (0,)), ((), ())), precision=prec)


def _prep_body(sptsT_ref, tptsT_ref, spts_ref, tpts_ref, RT_ref,
               Wf1_ref, bf1_ref, Wf2_ref, bf2_ref, wwl_ref,
               Wd1_ref, bd1_ref, Wd2_ref, bd2_ref, off_ref,
               keypts_ref, trans_ref, nn_ref, dfe_ref, table_ref):
    b = pl.program_id(0)
    sT = sptsT_ref[0]
    tT = tptsT_ref[0]
    sp = spts_ref[0]
    tp = tpts_ref[0]
    W1 = Wf1_ref[...]
    b1 = bf1_ref[...]
    W2 = Wf2_ref[...]
    b2 = bf2_ref[...]

    src_feat = jax.nn.relu(_mm(jax.nn.relu(_mm(sT, W1, _DEF) + b1), W2, _DEF) + b2)
    tgt_feat = jax.nn.relu(_mm(jax.nn.relu(_mm(tT, W1, _DEF) + b1), W2, _DEF) + b2)
    scores = _mm(src_feat, wwl_ref[...], _DEF)

    col_iota_n = lax.broadcasted_iota(jnp.int32, (N, 1), 0)
    lane_kn = lax.broadcasted_iota(jnp.int32, (K, N), 1)
    row_kn = lax.broadcasted_iota(jnp.int32, (K, N), 0)

    def topk_body(j, carry):
        sc, sel = carry
        m = jnp.max(sc)
        idx = jnp.min(jnp.where(sc == m, col_iota_n, N))
        sel = sel + jnp.where((row_kn == j) & (lane_kn == idx), 1.0, 0.0)
        sc = jnp.where(col_iota_n == idx, -jnp.inf, sc)
        return sc, sel

    _, sel = lax.fori_loop(0, K, topk_body,
                           (scores, jnp.zeros((K, N), jnp.float32)))

    keypts = _mm(sel, sT, _HI)
    key_feat = _mm(sel, src_feat, _HI)
    keyptsT = lax.dot_general(sp, sel, (((1,), (1,)), ((), ())),
                              precision=_HI)

    dx = keypts[:, 0:1] - keyptsT[0:1, :]
    dy = keypts[:, 1:2] - keyptsT[1:2, :]
    dz = keypts[:, 2:3] - keyptsT[2:3, :]
    d2 = (dx * dx + dy * dy) + dz * dz

    col_kk = lax.broadcasted_iota(jnp.int32, (K, K), 1)
    rowmod = lax.broadcasted_iota(jnp.int32, (K * NS, 1), 0) % NS

    def knn_body(s, carry):
        d2w, G = carry
        rm = jnp.min(d2w, axis=1, keepdims=True)
        idxc = jnp.min(jnp.where(d2w == rm, col_kk, K), axis=1, keepdims=True)
        oh = jnp.where(col_kk == idxc, 1.0, 0.0)
        exp_oh = jnp.broadcast_to(oh[:, None, :], (K, NS, K)).reshape(K * NS, K)
        G = G + jnp.where(rowmod == s, exp_oh, 0.0)
        d2w = jnp.where(col_kk == idxc, jnp.inf, d2w)
        return d2w, G

    _, G = lax.fori_loop(0, NS, knn_body,
                         (d2, jnp.zeros((K * NS, K), jnp.float32)))

    gxyz = _mm(G, keypts, _HI)
    gfeat = _mm(G, key_feat, _HI)
    kp_rep = jnp.broadcast_to(keypts[:, None, :], (K, NS, 3)).reshape(K * NS, 3)
    cat = jnp.concatenate([gxyz - kp_rep, gfeat], axis=1)
    hs = jax.nn.relu(_mm(cat, Wd1_ref[...], _DEF) + bd1_ref[...])
    hs = jax.nn.relu(_mm(hs, Wd2_ref[...], _DEF) + bd2_ref[...])
    dfe_ref[0] = jnp.max(hs.reshape(K, NS, F), axis=1)

    trans = _mm(keypts, RT_ref[...], _DEF)
    cand = trans[:, None, :] + off_ref[...][None, :, :]
    cf = cand.reshape(Q, 3)
    keypts_ref[0] = keypts
    trans_ref[0] = trans

    table_ref[0] = jnp.concatenate(
        [tT, tgt_feat, jnp.zeros((N, DT - 3 - F), jnp.float32)], axis=1)

    tx = tp[0:1, :]
    ty = tp[1:2, :]
    tz = tp[2:3, :]
    tn = (tx * tx + ty * ty) + tz * tz
    lane_qn = lax.broadcasted_iota(jnp.int32, (QCH, N), 1)
    boff = b * N
    for t in range(NCH):
        cfc = cf[t * QCH:(t + 1) * QCH, :]
        qn = (cfc[:, 0:1] * cfc[:, 0:1] + cfc[:, 1:2] * cfc[:, 1:2]) \
            + cfc[:, 2:3] * cfc[:, 2:3]
        cross = _mm(cfc, tp, _DEF)
        d2t = (qn + tn) - 2.0 * cross
        m = jnp.min(d2t, axis=1, keepdims=True)
        idx = jnp.min(jnp.where(d2t == m, lane_qn, N), axis=1, keepdims=True)
        nn_ref[0, t * QCH:(t + 1) * QCH, :] = idx + boff


KB = 16
QB = KB * C


def _cpg_body(g_ref, trans_ref, dfe_ref, off_ref, Wt1_ref, bt1_ref, Wt2_ref,
              bt2_ref, vcp_ref):
    g = g_ref[0]
    trans = trans_ref[0]
    cand = trans[:, None, :] + off_ref[...][None, :, :]
    cf = cand.reshape(QB, 3)
    tcat = jnp.concatenate([cf - g[:, 0:3], g[:, 3:3 + F]], axis=1)
    ht = jax.nn.relu(_mm(tcat, Wt1_ref[...], _DEF) + bt1_ref[...])
    ht = jax.nn.relu(_mm(ht, Wt2_ref[...], _DEF) + bt2_ref[...])
    dfe = dfe_ref[0]
    sim = jnp.sum(ht.reshape(KB, C, F) * dfe[:, None, :], axis=2)
    m = jnp.max(sim, axis=1, keepdims=True)
    e = jnp.exp(sim - m)
    w = e / jnp.sum(e, axis=1, keepdims=True)
    vcp_ref[0] = jnp.sum(w[:, :, None] * cand, axis=1)


_PREP_IN_SPECS = [
    pl.BlockSpec((1, N, 3), lambda b: (b, 0, 0)),
    pl.BlockSpec((1, N, 3), lambda b: (b, 0, 0)),
    pl.BlockSpec((1, 3, N), lambda b: (b, 0, 0)),
    pl.BlockSpec((1, 3, N), lambda b: (b, 0, 0)),
    pl.BlockSpec((3, 3), lambda b: (0, 0)),
    pl.BlockSpec((3, F), lambda b: (0, 0)),
    pl.BlockSpec((1, F), lambda b: (0, 0)),
    pl.BlockSpec((F, F), lambda b: (0, 0)),
    pl.BlockSpec((1, F), lambda b: (0, 0)),
    pl.BlockSpec((F, 1), lambda b: (0, 0)),
    pl.BlockSpec((3 + F, 64), lambda b: (0, 0)),
    pl.BlockSpec((1, 64), lambda b: (0, 0)),
    pl.BlockSpec((64, F), lambda b: (0, 0)),
    pl.BlockSpec((1, F), lambda b: (0, 0)),
    pl.BlockSpec((C, 3), lambda b: (0, 0)),
]
_PREP_OUT_SPECS = [
    pl.BlockSpec((1, K, 3), lambda b: (b, 0, 0)),
    pl.BlockSpec((1, K, 3), lambda b: (b, 0, 0)),
    pl.BlockSpec((1, Q, 1), lambda b: (b, 0, 0)),
    pl.BlockSpec((1, K, F), lambda b: (b, 0, 0)),
    pl.BlockSpec((1, N, DT), lambda b: (b, 0, 0)),
]
_PREP_OUT_SHAPES = [
    jax.ShapeDtypeStruct((B, K, 3), jnp.float32),
    jax.ShapeDtypeStruct((B, K, 3), jnp.float32),
    jax.ShapeDtypeStruct((B, Q, 1), jnp.int32),
    jax.ShapeDtypeStruct((B, K, F), jnp.float32),
    jax.ShapeDtypeStruct((B, N, DT), jnp.float32),
]

_CPG_IN_SPECS = [
    pl.BlockSpec((1, QB, DT), lambda b, kc: (b, kc, 0)),
    pl.BlockSpec((1, KB, 3), lambda b, kc: (b, kc, 0)),
    pl.BlockSpec((1, KB, F), lambda b, kc: (b, kc, 0)),
    pl.BlockSpec((C, 3), lambda b, kc: (0, 0)),
    pl.BlockSpec((3 + F, 64), lambda b, kc: (0, 0)),
    pl.BlockSpec((1, 64), lambda b, kc: (0, 0)),
    pl.BlockSpec((64, F), lambda b, kc: (0, 0)),
    pl.BlockSpec((1, F), lambda b, kc: (0, 0)),
]
_CPG_OUT_SPEC = pl.BlockSpec((1, KB, 3), lambda b, kc: (b, kc, 0))
_CPG_OUT_SHAPE = jax.ShapeDtypeStruct((B, K, 3), jnp.float32)


@functools.cache
def _make_sc_gather():
    @functools.partial(
        pl.kernel,
        out_type=jax.ShapeDtypeStruct((ROWS, DT), jnp.float32),
        mesh=plsc.VectorSubcoreMesh(core_axis_name="c", subcore_axis_name="s"),
        scratch_types=[
            pltpu.VMEM((RPW,), jnp.int32),
            pltpu.VMEM((RPW, DT), jnp.float32),
            pltpu.SemaphoreType.DMA,
        ],
    )
    def _sc_gather(table_hbm, idx_hbm, out_hbm, idx_v, rows_v, sem):
        wid = lax.axis_index("s") * 2 + lax.axis_index("c")
        base = wid * RPW
        pltpu.sync_copy(idx_hbm.at[pl.ds(base, RPW)], idx_v)
        copies = [
            pltpu.async_copy(table_hbm.at[idx_v.at[pl.ds(j * GCH, GCH)]],
                             rows_v.at[pl.ds(j * GCH, GCH)], sem)
            for j in range(NGC)
        ]
        for cp in copies:
            cp.wait()
        pltpu.sync_copy(rows_v, out_hbm.at[pl.ds(base, RPW)])

    return _sc_gather


def kernel(src_pts, tgt_pts, R_init, t_init, W_fe1, b_fe1, W_fe2, b_fe2,
           w_wl, W_ds1, b_ds1, W_ds2, b_ds2, W_dt1, b_dt1, W_dt2, b_dt2):
    sptsT = jnp.transpose(src_pts, (0, 2, 1))
    tptsT = jnp.transpose(tgt_pts, (0, 2, 1))
    g = jnp.linspace(-1.0, 1.0, 6)
    off = jnp.stack(jnp.meshgrid(g, g, g, indexing="ij"), -1).reshape(-1, 3)

    keypts, trans, nn, dfe, table = pl.pallas_call(
        _prep_body,
        grid=(B,),
        in_specs=_PREP_IN_SPECS,
        out_specs=_PREP_OUT_SPECS,
        out_shape=_PREP_OUT_SHAPES,
    )(sptsT, tptsT, src_pts, tgt_pts, R_init.T,
      W_fe1, b_fe1.reshape(1, F), W_fe2, b_fe2.reshape(1, F), w_wl,
      W_ds1, b_ds1.reshape(1, 64), W_ds2, b_ds2.reshape(1, F), off)

    gathered = _make_sc_gather()(table.reshape(B * N, DT), nn.reshape(ROWS))

    vcp = pl.pallas_call(
        _cpg_body,
        grid=(B, K // KB),
        in_specs=_CPG_IN_SPECS,
        out_specs=_CPG_OUT_SPEC,
        out_shape=_CPG_OUT_SHAPE,
    )(gathered.reshape(B, Q, DT), trans, dfe, off,
      W_dt1, b_dt1.reshape(1, 64), W_dt2, b_dt2.reshape(1, F))

    return keypts, vcp
